# baseline (speedup 1.0000x reference)
import functools

import jax
import jax.numpy as jnp
from jax import lax
from jax.experimental import pallas as pl
from jax.experimental.pallas import tpu as pltpu
from jax.experimental.pallas import tpu_sc as plsc

N_INST = 10000
N_AUTH = 100000
N_PAPER = 100000
N_FIELD = 50000
E_IA = 400000
E_AP = 1600000
E_PP = 1600000
E_PF = 1600000
D_FEAT = 128

_NC, _NS = 2, 16
_NW = _NC * _NS
_KB = 25
_NBLK_BIG = E_PP // (128 * _KB)
_NBLK_IA = E_IA // (128 * _KB)

_NP_P = 102400
_SL_P = _NP_P // _NS
_NBW = _NP_P // 32
_WPT = _NBW // _NS

_MESH = plsc.VectorSubcoreMesh(
    core_axis_name="c", subcore_axis_name="s", num_cores=_NC, num_subcores=_NS)
_SC_PARAMS = pltpu.CompilerParams(
    use_tc_tiling_on_sc=False, needs_layout_passes=False)

_f32 = jnp.float32
_i32 = jnp.int32


def _wid():
    c = lax.axis_index("c")
    s = lax.axis_index("s")
    return c, s, s * _NC + c


def _nblk_for(wid, nblk):
    return (nblk - wid + _NW - 1) // _NW


def _mega(pf0r, pp1r, ia1r, ap0r, ap1r):
    out_type = [
        jax.ShapeDtypeStruct((_NC, _NP_P), _f32),
        jax.ShapeDtypeStruct((_NC, _NP_P), _f32),
        jax.ShapeDtypeStruct((_NC, _NP_P), _f32),
        jax.ShapeDtypeStruct((_NP_P,), _f32),
        jax.ShapeDtypeStruct((_NC, _NS, 16), _f32),
    ]
    scratch = [
        pltpu.VMEM((_KB, 128), _i32),
        pltpu.VMEM((128,), _f32),
        pltpu.VMEM_SHARED((_NP_P,), _f32),
        pltpu.VMEM_SHARED((_NP_P,), _f32),
        pltpu.VMEM_SHARED((_NP_P,), _f32),
        pltpu.VMEM((_SL_P,), _f32),
        pltpu.VMEM((_SL_P,), _f32),
        pltpu.VMEM((_SL_P,), _f32),
        pltpu.VMEM((_WPT,), _i32),
        pltpu.VMEM((_NBW,), _i32),
        pltpu.VMEM((_NBW,), _i32),
        pltpu.VMEM((_KB, 128), _i32),
        pltpu.VMEM((_KB, 128), _i32),
        pltpu.VMEM((16,), _f32),
        pltpu.VMEM_SHARED((_NBW,), _i32),
        pltpu.VMEM_SHARED((_NBW,), _i32),
        pltpu.SemaphoreType.DMA,
        pltpu.SemaphoreType.REGULAR,
    ]

    @functools.partial(pl.kernel, out_type=out_type, mesh=_MESH,
                       scratch_types=scratch, compiler_params=_SC_PARAMS)
    def k(pf_h, pp_h, ia_h, ap0_h, ap1_h, ones_h, zeros_h,
          o_pf, o_pp, o_ia, o_pb, o_rch,
          idx_v, ones_v, acc_pf, acc_pp, acc_ia,
          b0, b1, bc, wv, ahb_v, plb_v, a_v, p_v, out_v,
          ahb_sh, plb_sh, sem, csem):
        c, s, wid = _wid()
        sl = pl.ds(s * _SL_P, _SL_P)
        lanes = lax.iota(_i32, 16)

        pltpu.sync_copy(ones_h, ones_v)
        pltpu.sync_copy(zeros_h, acc_pf.at[sl])
        pltpu.sync_copy(zeros_h, acc_pp.at[sl])
        pltpu.sync_copy(zeros_h, acc_ia.at[sl])
        plsc.subcore_barrier()

        def run(e_h, nblk, acc):
            def body(i, carry):
                blk = wid + i * _NW
                pltpu.sync_copy(e_h.at[pl.ds(blk * _KB, _KB)], idx_v)
                cps = [pltpu.async_copy(ones_v, acc.at[idx_v.at[j]], sem,
                                        add=True)
                       for j in range(_KB)]
                for cp in cps:
                    cp.wait()
                return carry
            lax.fori_loop(0, _nblk_for(wid, nblk), body, 0)

        run(pf_h, _NBLK_BIG, acc_pf)
        run(pp_h, _NBLK_BIG, acc_pp)
        run(ia_h, _NBLK_IA, acc_ia)
        plsc.subcore_barrier()

        pltpu.sync_copy(acc_pf.at[sl], o_pf.at[c, sl])
        pltpu.sync_copy(acc_pp.at[sl], o_pp.at[c, sl])
        pltpu.sync_copy(acc_ia.at[sl], o_ia.at[c, sl])
        plsc.subcore_barrier()
        pltpu.core_barrier(csem, core_axis_name="c")

        def combine(h2, dst):
            pltpu.sync_copy(h2.at[0, sl], b0)
            pltpu.sync_copy(h2.at[1, sl], b1)

            def body(i, carry):
                i16 = pl.ds(i * 16, 16)
                dst[i16] = b0[i16] + b1[i16]
                return carry
            lax.fori_loop(0, _SL_P // 16, body, 0)

        def pack_bits(src, bits_sh):
            for base in [*range(0, _WPT - 15, 16), _WPT - 16]:
                cur = jnp.zeros((16,), _i32)
                for t in range(16):
                    w = base + t
                    lo = (src[pl.ds(w * 32, 16)] > 0).astype(_i32) << lanes
                    hi = (src[pl.ds(w * 32 + 16, 16)] > 0).astype(_i32) << lanes
                    wd = jnp.sum(lo) | (jnp.sum(hi) << 16)
                    cur = jnp.where(lanes == t, wd, cur)
                wv[pl.ds(base, 16)] = cur
            pltpu.sync_copy(wv, bits_sh.at[pl.ds(s * _WPT, _WPT)])

        combine(o_pf, bc)
        pack_bits(bc, plb_sh)
        pltpu.sync_copy(o_pp.at[0, sl], b0)
        pltpu.sync_copy(o_pp.at[1, sl], b1)

        def pb_body(i, carry):
            i16 = pl.ds(i * 16, 16)
            pos = (bc[i16] > 0).astype(_f32)
            b0[i16] = pos * (b0[i16] + b1[i16])
            return carry
        lax.fori_loop(0, _SL_P // 16, pb_body, 0)

        @pl.when(c == 0)
        def _():
            pltpu.sync_copy(b0, o_pb.at[sl])

        combine(o_ia, bc)
        pack_bits(bc, ahb_sh)
        plsc.subcore_barrier()
        pltpu.sync_copy(ahb_sh, ahb_v)
        pltpu.sync_copy(plb_sh, plb_v)

        def bit_at(tab, idx16):
            w = plsc.load_gather(tab, [lax.shift_right_logical(idx16, 5)])
            return lax.shift_right_logical(w, idx16 & 31) & 1

        def body(i, acc):
            blk = wid + i * _NW
            pltpu.sync_copy(ap0_h.at[pl.ds(blk * _KB, _KB)], a_v)
            pltpu.sync_copy(ap1_h.at[pl.ds(blk * _KB, _KB)], p_v)
            for j in range(_KB):
                for i2 in range(8):
                    sl16 = pl.ds(i2 * 16, 16)
                    hit = bit_at(ahb_v, a_v[j, sl16]) & bit_at(plb_v, p_v[j, sl16])
                    acc = acc + hit.astype(_f32)
            return acc

        acc = lax.fori_loop(0, _nblk_for(wid, _NBLK_BIG), body,
                            jnp.zeros((16,), _f32))
        out_v[...] = acc
        pltpu.sync_copy(out_v, o_rch.at[c, s])

    ones = jnp.ones((128,), _f32)
    zeros = jnp.zeros((_SL_P,), _f32)
    return k(pf0r, pp1r, ia1r, ap0r, ap1r, ones, zeros)


_RB = 10000


def _modulate(x, pb_pad, reach_part):
    def body(x_ref, pb_ref, pb2_ref, rch_ref, o_ref, s_ref):
        i = pl.program_id(0)

        @pl.when(i == 0)
        def _():
            ps = jnp.sum(pb2_ref[...])
            rs = jnp.sum(rch_ref[...])
            s_ref[0] = 1.0 / jnp.maximum(ps, 1e-12)
            s_ref[1] = (jnp.where(ps > 0, 1.0, 0.0)
                        + jnp.where(rs > 0, 1.0, 0.0))

        o_ref[...] = x_ref[...] * (pb_ref[...] * s_ref[0]) + s_ref[1]

    pb2 = pb_pad.reshape(_NP_P // 128, 128)
    rch = reach_part.reshape(_NC * _NS, 16)
    return pl.pallas_call(
        body,
        grid=(N_PAPER // _RB,),
        in_specs=[
            pl.BlockSpec((_RB, D_FEAT), lambda i: (i, 0)),
            pl.BlockSpec((_RB, 1), lambda i: (i, 0)),
            pl.BlockSpec(pb2.shape, lambda i: (0, 0)),
            pl.BlockSpec(rch.shape, lambda i: (0, 0)),
        ],
        out_specs=pl.BlockSpec((_RB, D_FEAT), lambda i: (i, 0)),
        out_shape=jax.ShapeDtypeStruct((N_PAPER, D_FEAT), jnp.float32),
        scratch_shapes=[pltpu.SMEM((2,), jnp.float32)],
    )(x, pb_pad[:N_PAPER].reshape(N_PAPER, 1), pb2, rch)


def kernel(x_paper, edge_inst_auth, edge_auth_paper, edge_cite, edge_paper_field):
    r = lambda e: e.reshape(-1, 128)
    pf0r = r(edge_paper_field[0])
    pp1r = r(edge_cite[1])
    ia1r = r(edge_inst_auth[1])
    ap0r, ap1r = r(edge_auth_paper[0]), r(edge_auth_paper[1])

    _, _, _, pb_pad, reach_part = _mega(pf0r, pp1r, ia1r, ap0r, ap1r)
    return _modulate(x_paper, pb_pad, reach_part)


# speedup vs baseline: 1.1439x; 1.1439x over previous
import functools

import jax
import jax.numpy as jnp
from jax import lax
from jax.experimental import pallas as pl
from jax.experimental.pallas import tpu as pltpu
from jax.experimental.pallas import tpu_sc as plsc

N_INST = 10000
N_AUTH = 100000
N_PAPER = 100000
N_FIELD = 50000
E_IA = 400000
E_AP = 1600000
E_PP = 1600000
E_PF = 1600000
D_FEAT = 128

_NC, _NS = 2, 16
_NW = _NC * _NS
_KB = 25
_NBLK_BIG = E_PP // (128 * _KB)
_NBLK_IA = E_IA // (128 * _KB)

_NP_P = 102400
_SL_P = _NP_P // _NS
_NBW = _NP_P // 32
_WPT = _NBW // _NS

_MESH = plsc.VectorSubcoreMesh(
    core_axis_name="c", subcore_axis_name="s", num_cores=_NC, num_subcores=_NS)
_SC_PARAMS = pltpu.CompilerParams(
    use_tc_tiling_on_sc=False, needs_layout_passes=False)

_f32 = jnp.float32
_i32 = jnp.int32


def _wid():
    c = lax.axis_index("c")
    s = lax.axis_index("s")
    return c, s, s * _NC + c


def _nblk_for(wid, nblk):
    return (nblk - wid + _NW - 1) // _NW


def _hist3(pf0r, pp1r, ia1r):
    out_type = [jax.ShapeDtypeStruct((_NC, _NP_P), _f32) for _ in range(3)]
    scratch = [
        pltpu.VMEM((_KB * 128,), _i32),
        pltpu.VMEM((128,), _f32),
        pltpu.VMEM_SHARED((_NP_P,), _f32),
        pltpu.VMEM_SHARED((_NP_P,), _f32),
        pltpu.VMEM_SHARED((_NP_P,), _f32),
        pltpu.SemaphoreType.DMA,
    ]

    @functools.partial(pl.kernel, out_type=out_type, mesh=_MESH,
                       scratch_types=scratch, compiler_params=_SC_PARAMS)
    def k(pf_h, pp_h, ia_h, ones_h, zeros_h,
          o_pf, o_pp, o_ia, idx_v, ones_v, acc_pf, acc_pp, acc_ia, sem):
        c, s, wid = _wid()
        sl = pl.ds(s * _SL_P, _SL_P)
        pltpu.sync_copy(ones_h, ones_v)
        pltpu.sync_copy(zeros_h, acc_pf.at[sl])
        pltpu.sync_copy(zeros_h, acc_pp.at[sl])
        pltpu.sync_copy(zeros_h, acc_ia.at[sl])
        plsc.subcore_barrier()

        def run(e_h, nblk, acc):
            def body(i, carry):
                blk = wid + i * _NW
                pltpu.sync_copy(e_h.at[pl.ds(blk * _KB * 128, _KB * 128)],
                                idx_v)
                cps = [pltpu.async_copy(
                    ones_v, acc.at[idx_v.at[pl.ds(j * 128, 128)]], sem,
                    add=True)
                       for j in range(_KB)]
                for cp in cps:
                    cp.wait()
                return carry
            lax.fori_loop(0, _nblk_for(wid, nblk), body, 0)

        run(pf_h, _NBLK_BIG, acc_pf)
        run(pp_h, _NBLK_BIG, acc_pp)
        run(ia_h, _NBLK_IA, acc_ia)
        plsc.subcore_barrier()
        pltpu.sync_copy(acc_pf.at[sl], o_pf.at[c, sl])
        pltpu.sync_copy(acc_pp.at[sl], o_pp.at[c, sl])
        pltpu.sync_copy(acc_ia.at[sl], o_ia.at[c, sl])

    ones = jnp.ones((128,), _f32)
    zeros = jnp.zeros((_SL_P,), _f32)
    return k(pf0r, pp1r, ia1r, ones, zeros)


def _build_reach(h_pf, h_pp, h_ia, ap0r, ap1r):
    out_type = [
        jax.ShapeDtypeStruct((_NP_P,), _f32),
        jax.ShapeDtypeStruct((_NC, _NS, 16), _f32),
    ]
    scratch = [
        pltpu.VMEM((_SL_P,), _f32),
        pltpu.VMEM((_SL_P,), _f32),
        pltpu.VMEM((_SL_P,), _f32),
        pltpu.VMEM((_WPT,), _i32),
        pltpu.VMEM((_NBW,), _i32),
        pltpu.VMEM((_NBW,), _i32),
        pltpu.VMEM((_KB * 128,), _i32),
        pltpu.VMEM((_KB * 128,), _i32),
        pltpu.VMEM((16,), _f32),
        pltpu.VMEM_SHARED((_NBW,), _i32),
        pltpu.VMEM_SHARED((_NBW,), _i32),
    ]

    @functools.partial(pl.kernel, out_type=out_type, mesh=_MESH,
                       scratch_types=scratch, compiler_params=_SC_PARAMS)
    def k(pf_h, pp_h, ia_h, ap0_h, ap1_h, o_pb, o_rch,
          b0, b1, bc, wv, ahb_v, plb_v, a_v, p_v, out_v, ahb_sh, plb_sh):
        c, s, wid = _wid()
        sl = pl.ds(s * _SL_P, _SL_P)
        lanes = lax.iota(_i32, 16)

        def combine(h2, dst):
            pltpu.sync_copy(h2.at[0, sl], b0)
            pltpu.sync_copy(h2.at[1, sl], b1)

            def body(i, carry):
                i16 = pl.ds(i * 16, 16)
                dst[i16] = b0[i16] + b1[i16]
                return carry
            lax.fori_loop(0, _SL_P // 16, body, 0)

        def pack_bits(src, bits_sh):
            for base in [*range(0, _WPT - 15, 16), _WPT - 16]:
                cur = jnp.zeros((16,), _i32)
                for t in range(16):
                    w = base + t
                    lo = (src[pl.ds(w * 32, 16)] > 0).astype(_i32) << lanes
                    hi = (src[pl.ds(w * 32 + 16, 16)] > 0).astype(_i32) << lanes
                    wd = jnp.sum(lo) | (jnp.sum(hi) << 16)
                    cur = jnp.where(lanes == t, wd, cur)
                wv[pl.ds(base, 16)] = cur
            pltpu.sync_copy(wv, bits_sh.at[pl.ds(s * _WPT, _WPT)])

        combine(pf_h, bc)
        pack_bits(bc, plb_sh)
        pltpu.sync_copy(pp_h.at[0, sl], b0)
        pltpu.sync_copy(pp_h.at[1, sl], b1)

        def pb_body(i, carry):
            i16 = pl.ds(i * 16, 16)
            pos = (bc[i16] > 0).astype(_f32)
            b0[i16] = pos * (b0[i16] + b1[i16])
            return carry
        lax.fori_loop(0, _SL_P // 16, pb_body, 0)

        @pl.when(c == 0)
        def _():
            pltpu.sync_copy(b0, o_pb.at[sl])

        combine(ia_h, bc)
        pack_bits(bc, ahb_sh)
        plsc.subcore_barrier()
        pltpu.sync_copy(ahb_sh, ahb_v)
        pltpu.sync_copy(plb_sh, plb_v)

        def bit_at(tab, idx16):
            w = plsc.load_gather(tab, [lax.shift_right_logical(idx16, 5)])
            return lax.shift_right_logical(w, idx16 & 31) & 1

        def body(i, acc):
            blk = wid + i * _NW
            pltpu.sync_copy(ap0_h.at[pl.ds(blk * _KB * 128, _KB * 128)], a_v)
            pltpu.sync_copy(ap1_h.at[pl.ds(blk * _KB * 128, _KB * 128)], p_v)
            for j in range(_KB * 8):
                sl16 = pl.ds(j * 16, 16)
                hit = bit_at(ahb_v, a_v[sl16]) & bit_at(plb_v, p_v[sl16])
                acc = acc + hit.astype(_f32)
            return acc

        acc = lax.fori_loop(0, _nblk_for(wid, _NBLK_BIG), body,
                            jnp.zeros((16,), _f32))
        out_v[...] = acc
        pltpu.sync_copy(out_v, o_rch.at[c, s])

    return k(h_pf, h_pp, h_ia, ap0r, ap1r)


_RB = 10000


def _modulate(x, pb_pad, reach_part):
    def body(x_ref, pb_ref, pb2_ref, rch_ref, o_ref, s_ref):
        i = pl.program_id(0)

        @pl.when(i == 0)
        def _():
            ps = jnp.sum(pb2_ref[...])
            rs = jnp.sum(rch_ref[...])
            s_ref[0] = 1.0 / jnp.maximum(ps, 1e-12)
            s_ref[1] = (jnp.where(ps > 0, 1.0, 0.0)
                        + jnp.where(rs > 0, 1.0, 0.0))

        o_ref[...] = x_ref[...] * (pb_ref[...] * s_ref[0]) + s_ref[1]

    pb2 = pb_pad.reshape(_NP_P // 128, 128)
    rch = reach_part.reshape(_NC * _NS, 16)
    return pl.pallas_call(
        body,
        grid=(N_PAPER // _RB,),
        in_specs=[
            pl.BlockSpec((_RB, D_FEAT), lambda i: (i, 0)),
            pl.BlockSpec((_RB, 1), lambda i: (i, 0)),
            pl.BlockSpec(pb2.shape, lambda i: (0, 0)),
            pl.BlockSpec(rch.shape, lambda i: (0, 0)),
        ],
        out_specs=pl.BlockSpec((_RB, D_FEAT), lambda i: (i, 0)),
        out_shape=jax.ShapeDtypeStruct((N_PAPER, D_FEAT), jnp.float32),
        scratch_shapes=[pltpu.SMEM((2,), jnp.float32)],
    )(x, pb_pad[:N_PAPER].reshape(N_PAPER, 1), pb2, rch)


def kernel(x_paper, edge_inst_auth, edge_auth_paper, edge_cite, edge_paper_field):
    h_pf, h_pp1, h_ia1 = _hist3(
        edge_paper_field[0], edge_cite[1], edge_inst_auth[1])
    pb_pad, reach_part = _build_reach(
        h_pf, h_pp1, h_ia1, edge_auth_paper[0], edge_auth_paper[1])
    return _modulate(x_paper, pb_pad, reach_part)


# speedup vs baseline: 1.4689x; 1.2842x over previous
import functools

import jax
import jax.numpy as jnp
from jax import lax
from jax.experimental import pallas as pl
from jax.experimental.pallas import tpu as pltpu
from jax.experimental.pallas import tpu_sc as plsc

N_INST = 10000
N_AUTH = 100000
N_PAPER = 100000
N_FIELD = 50000
E_IA = 400000
E_AP = 1600000
E_PP = 1600000
E_PF = 1600000
D_FEAT = 128

_NC, _NS = 2, 16
_NW = _NC * _NS
_KB = 25
_NBLK_BIG = E_PP // (128 * _KB)
_NBLK_IA = E_IA // (128 * _KB)

_NP_P = 102400
_SL_P = _NP_P // _NS
_NBW = _NP_P // 32
_WPT = _NBW // _NS

_MESH = plsc.VectorSubcoreMesh(
    core_axis_name="c", subcore_axis_name="s", num_cores=_NC, num_subcores=_NS)
_SC_PARAMS = pltpu.CompilerParams(
    use_tc_tiling_on_sc=False, needs_layout_passes=False)

_f32 = jnp.float32
_i32 = jnp.int32


def _wid():
    c = lax.axis_index("c")
    s = lax.axis_index("s")
    return c, s, s * _NC + c


def _nblk_for(wid, nblk):
    return (nblk - wid + _NW - 1) // _NW


def _hist3(pf0r, pp1r, ia1r):
    out_type = [jax.ShapeDtypeStruct((_NC, _NP_P), _f32) for _ in range(3)]
    scratch = [
        pltpu.VMEM((_KB * 128,), _i32),
        pltpu.VMEM((128,), _f32),
        pltpu.VMEM_SHARED((_NP_P,), _f32),
        pltpu.VMEM_SHARED((_NP_P,), _f32),
        pltpu.VMEM_SHARED((_NP_P,), _f32),
        pltpu.SemaphoreType.DMA,
    ]

    @functools.partial(pl.kernel, out_type=out_type, mesh=_MESH,
                       scratch_types=scratch, compiler_params=_SC_PARAMS)
    def k(pf_h, pp_h, ia_h, ones_h, zeros_h,
          o_pf, o_pp, o_ia, idx_v, ones_v, acc_pf, acc_pp, acc_ia, sem):
        c, s, wid = _wid()
        sl = pl.ds(s * _SL_P, _SL_P)
        pltpu.sync_copy(ones_h, ones_v)
        pltpu.sync_copy(zeros_h, acc_pf.at[sl])
        pltpu.sync_copy(zeros_h, acc_pp.at[sl])
        pltpu.sync_copy(zeros_h, acc_ia.at[sl])
        plsc.subcore_barrier()

        def run(e_h, row, nblk, acc):
            def body(i, carry):
                blk = wid + i * _NW
                pltpu.sync_copy(e_h.at[row, pl.ds(blk * _KB * 128, _KB * 128)],
                                idx_v)
                cps = [pltpu.async_copy(
                    ones_v, acc.at[idx_v.at[pl.ds(j * 128, 128)]], sem,
                    add=True)
                       for j in range(_KB)]
                for cp in cps:
                    cp.wait()
                return carry
            lax.fori_loop(0, _nblk_for(wid, nblk), body, 0)

        run(pf_h, 0, _NBLK_BIG, acc_pf)
        run(pp_h, 1, _NBLK_BIG, acc_pp)
        run(ia_h, 1, _NBLK_IA, acc_ia)
        plsc.subcore_barrier()
        pltpu.sync_copy(acc_pf.at[sl], o_pf.at[c, sl])
        pltpu.sync_copy(acc_pp.at[sl], o_pp.at[c, sl])
        pltpu.sync_copy(acc_ia.at[sl], o_ia.at[c, sl])

    ones = jnp.ones((128,), _f32)
    zeros = jnp.zeros((_SL_P,), _f32)
    return k(pf0r, pp1r, ia1r, ones, zeros)


def _build_reach(h_pf, h_pp, h_ia, ap_e):
    out_type = [
        jax.ShapeDtypeStruct((_NP_P,), _f32),
        jax.ShapeDtypeStruct((_NC, _NS, 16), _f32),
    ]
    scratch = [
        pltpu.VMEM((_SL_P,), _f32),
        pltpu.VMEM((_SL_P,), _f32),
        pltpu.VMEM((_SL_P,), _f32),
        pltpu.VMEM((_WPT,), _i32),
        pltpu.VMEM((_NBW,), _i32),
        pltpu.VMEM((_NBW,), _i32),
        pltpu.VMEM((_KB * 128,), _i32),
        pltpu.VMEM((_KB * 128,), _i32),
        pltpu.VMEM((16,), _f32),
        pltpu.VMEM_SHARED((_NBW,), _i32),
        pltpu.VMEM_SHARED((_NBW,), _i32),
    ]

    @functools.partial(pl.kernel, out_type=out_type, mesh=_MESH,
                       scratch_types=scratch, compiler_params=_SC_PARAMS)
    def k(pf_h, pp_h, ia_h, ap_h, o_pb, o_rch,
          b0, b1, bc, wv, ahb_v, plb_v, a_v, p_v, out_v, ahb_sh, plb_sh):
        c, s, wid = _wid()
        sl = pl.ds(s * _SL_P, _SL_P)
        lanes = lax.iota(_i32, 16)

        def combine(h2, dst):
            pltpu.sync_copy(h2.at[0, sl], b0)
            pltpu.sync_copy(h2.at[1, sl], b1)

            def body(i, carry):
                i16 = pl.ds(i * 16, 16)
                dst[i16] = b0[i16] + b1[i16]
                return carry
            lax.fori_loop(0, _SL_P // 16, body, 0)

        def pack_bits(src, bits_sh):
            for base in [*range(0, _WPT - 15, 16), _WPT - 16]:
                cur = jnp.zeros((16,), _i32)
                for t in range(16):
                    w = base + t
                    lo = (src[pl.ds(w * 32, 16)] > 0).astype(_i32) << lanes
                    hi = (src[pl.ds(w * 32 + 16, 16)] > 0).astype(_i32) << lanes
                    wd = jnp.sum(lo) | (jnp.sum(hi) << 16)
                    cur = jnp.where(lanes == t, wd, cur)
                wv[pl.ds(base, 16)] = cur
            pltpu.sync_copy(wv, bits_sh.at[pl.ds(s * _WPT, _WPT)])

        combine(pf_h, bc)
        pack_bits(bc, plb_sh)
        pltpu.sync_copy(pp_h.at[0, sl], b0)
        pltpu.sync_copy(pp_h.at[1, sl], b1)

        def pb_body(i, carry):
            i16 = pl.ds(i * 16, 16)
            pos = (bc[i16] > 0).astype(_f32)
            b0[i16] = pos * (b0[i16] + b1[i16])
            return carry
        lax.fori_loop(0, _SL_P // 16, pb_body, 0)

        @pl.when(c == 0)
        def _():
            pltpu.sync_copy(b0, o_pb.at[sl])

        combine(ia_h, bc)
        pack_bits(bc, ahb_sh)
        plsc.subcore_barrier()
        pltpu.sync_copy(ahb_sh, ahb_v)
        pltpu.sync_copy(plb_sh, plb_v)

        def bit_at(tab, idx16):
            w = plsc.load_gather(tab, [lax.shift_right_logical(idx16, 5)])
            return lax.shift_right_logical(w, idx16 & 31) & 1

        def body(i, acc):
            blk = wid + i * _NW
            sle = pl.ds(blk * _KB * 128, _KB * 128)
            pltpu.sync_copy(ap_h.at[0, sle], a_v)
            pltpu.sync_copy(ap_h.at[1, sle], p_v)
            for j in range(_KB * 8):
                sl16 = pl.ds(j * 16, 16)
                hit = bit_at(ahb_v, a_v[sl16]) & bit_at(plb_v, p_v[sl16])
                acc = acc + hit.astype(_f32)
            return acc

        acc = lax.fori_loop(0, _nblk_for(wid, _NBLK_BIG), body,
                            jnp.zeros((16,), _f32))
        out_v[...] = acc
        pltpu.sync_copy(out_v, o_rch.at[c, s])

    return k(h_pf, h_pp, h_ia, ap_e)


_RB = 10000


def _modulate(x, pb_pad, reach_part):
    def body(x_ref, pb_ref, pb2_ref, rch_ref, o_ref, s_ref):
        i = pl.program_id(0)

        @pl.when(i == 0)
        def _():
            ps = jnp.sum(pb2_ref[...])
            rs = jnp.sum(rch_ref[...])
            s_ref[0] = 1.0 / jnp.maximum(ps, 1e-12)
            s_ref[1] = (jnp.where(ps > 0, 1.0, 0.0)
                        + jnp.where(rs > 0, 1.0, 0.0))

        o_ref[...] = x_ref[...] * (pb_ref[...] * s_ref[0]) + s_ref[1]

    pb2 = pb_pad.reshape(_NP_P // 128, 128)
    rch = reach_part.reshape(_NC * _NS, 16)
    return pl.pallas_call(
        body,
        grid=(N_PAPER // _RB,),
        in_specs=[
            pl.BlockSpec((_RB, D_FEAT), lambda i: (i, 0)),
            pl.BlockSpec((_RB, 1), lambda i: (i, 0)),
            pl.BlockSpec(pb2.shape, lambda i: (0, 0)),
            pl.BlockSpec(rch.shape, lambda i: (0, 0)),
        ],
        out_specs=pl.BlockSpec((_RB, D_FEAT), lambda i: (i, 0)),
        out_shape=jax.ShapeDtypeStruct((N_PAPER, D_FEAT), jnp.float32),
        scratch_shapes=[pltpu.SMEM((2,), jnp.float32)],
    )(x, pb_pad[:N_PAPER].reshape(N_PAPER, 1), pb2, rch)


def kernel(x_paper, edge_inst_auth, edge_auth_paper, edge_cite, edge_paper_field):
    h_pf, h_pp1, h_ia1 = _hist3(
        edge_paper_field, edge_cite, edge_inst_auth)
    pb_pad, reach_part = _build_reach(h_pf, h_pp1, h_ia1, edge_auth_paper)
    return _modulate(x_paper, pb_pad, reach_part)


# speedup vs baseline: 1.7696x; 1.2047x over previous
import functools

import jax
import jax.numpy as jnp
from jax import lax
from jax.experimental import pallas as pl
from jax.experimental.pallas import tpu as pltpu
from jax.experimental.pallas import tpu_sc as plsc

N_INST = 10000
N_AUTH = 100000
N_PAPER = 100000
N_FIELD = 50000
E_IA = 400000
E_AP = 1600000
E_PP = 1600000
E_PF = 1600000
D_FEAT = 128

_NC, _NS = 2, 16
_NW = _NC * _NS
_CB = 3200
_NBLK_BIG = E_PP // _CB
_NBLK_IA = E_IA // _CB

_NP_P = 102400
_SL_P = _NP_P // _NS
_NBW = _NP_P // 32
_WPT = _NBW // _NS

_MESH = plsc.VectorSubcoreMesh(
    core_axis_name="c", subcore_axis_name="s", num_cores=_NC, num_subcores=_NS)
_SC_PARAMS = pltpu.CompilerParams(
    use_tc_tiling_on_sc=False, needs_layout_passes=False)

_f32 = jnp.float32
_i32 = jnp.int32


def _wid():
    c = lax.axis_index("c")
    s = lax.axis_index("s")
    return c, s, s * _NC + c


def _nblk_for(wid, nblk):
    return (nblk - wid + _NW - 1) // _NW


def _hist_rows(specs):
    n = len(specs)
    out_type = [jax.ShapeDtypeStruct((_NC, _NP_P), _f32) for _ in range(n)]
    scratch = ([pltpu.VMEM((_CB,), _i32), pltpu.VMEM((128,), _f32)]
               + [pltpu.VMEM_SHARED((_NP_P,), _f32) for _ in range(n)]
               + [pltpu.SemaphoreType.DMA])

    @functools.partial(pl.kernel, out_type=out_type, mesh=_MESH,
                       scratch_types=scratch, compiler_params=_SC_PARAMS)
    def k(*refs):
        e_hs = refs[:n]
        ones_h, zeros_h = refs[n], refs[n + 1]
        outs = refs[n + 2:n + 2 + n]
        idx_v, ones_v = refs[n + 2 + n], refs[n + 3 + n]
        accs = refs[n + 4 + n:n + 4 + 2 * n]
        sem = refs[-1]
        c, s, wid = _wid()
        sl = pl.ds(s * _SL_P, _SL_P)
        pltpu.sync_copy(ones_h, ones_v)
        for acc in accs:
            pltpu.sync_copy(zeros_h, acc.at[sl])
        plsc.subcore_barrier()

        for (row, nblk), e_h, acc in zip(specs, e_hs, accs):
            def body(i, carry, e_h=e_h, acc=acc, row=row):
                blk = wid + i * _NW
                pltpu.sync_copy(e_h.at[row, pl.ds(blk * _CB, _CB)], idx_v)
                cps = [pltpu.async_copy(
                    ones_v, acc.at[idx_v.at[pl.ds(j * 128, 128)]], sem,
                    add=True)
                       for j in range(_CB // 128)]
                for cp in cps:
                    cp.wait()
                return carry
            lax.fori_loop(0, _nblk_for(wid, nblk), body, 0)

        plsc.subcore_barrier()
        for acc, o in zip(accs, outs):
            pltpu.sync_copy(acc.at[sl], o.at[c, sl])

    ones = jnp.ones((128,), _f32)
    zeros = jnp.zeros((_SL_P,), _f32)

    def call(*edge_arrays):
        return k(*edge_arrays, ones, zeros)
    return call


_hist_pf = _hist_rows([(0, _NBLK_BIG)])
_hist_ppia = _hist_rows([(1, _NBLK_BIG), (1, _NBLK_IA)])


def _build(h_pf, h_pp, h_ia):
    out_type = [
        jax.ShapeDtypeStruct((_NP_P,), _f32),
        jax.ShapeDtypeStruct((_NBW,), _i32),
        jax.ShapeDtypeStruct((_NBW,), _i32),
    ]
    scratch = [
        pltpu.VMEM((_SL_P,), _f32),
        pltpu.VMEM((_SL_P,), _f32),
        pltpu.VMEM((_SL_P,), _f32),
        pltpu.VMEM((_WPT,), _i32),
    ]

    @functools.partial(pl.kernel, out_type=out_type, mesh=_MESH,
                       scratch_types=scratch, compiler_params=_SC_PARAMS)
    def k(pf_h, pp_h, ia_h, o_pb, o_plb, o_ahb, b0, b1, bc, wv):
        c, s, wid = _wid()
        sl = pl.ds(s * _SL_P, _SL_P)
        lanes = lax.iota(_i32, 16)

        def combine(h2, dst):
            pltpu.sync_copy(h2.at[0, sl], b0)
            pltpu.sync_copy(h2.at[1, sl], b1)

            def body(i, carry):
                i16 = pl.ds(i * 16, 16)
                dst[i16] = b0[i16] + b1[i16]
                return carry
            lax.fori_loop(0, _SL_P // 16, body, 0)

        def pack_bits(src, o_bits):
            for base in [*range(0, _WPT - 15, 16), _WPT - 16]:
                cur = jnp.zeros((16,), _i32)
                for t in range(16):
                    w = base + t
                    lo = (src[pl.ds(w * 32, 16)] > 0).astype(_i32) << lanes
                    hi = (src[pl.ds(w * 32 + 16, 16)] > 0).astype(_i32) << lanes
                    wd = jnp.sum(lo) | (jnp.sum(hi) << 16)
                    cur = jnp.where(lanes == t, wd, cur)
                wv[pl.ds(base, 16)] = cur

            @pl.when(c == 0)
            def _():
                pltpu.sync_copy(wv, o_bits.at[pl.ds(s * _WPT, _WPT)])

        combine(pf_h, bc)
        pack_bits(bc, o_plb)
        pltpu.sync_copy(pp_h.at[0, sl], b0)
        pltpu.sync_copy(pp_h.at[1, sl], b1)

        def pb_body(i, carry):
            i16 = pl.ds(i * 16, 16)
            pos = (bc[i16] > 0).astype(_f32)
            b0[i16] = pos * (b0[i16] + b1[i16])
            return carry
        lax.fori_loop(0, _SL_P // 16, pb_body, 0)

        @pl.when(c == 0)
        def _():
            pltpu.sync_copy(b0, o_pb.at[sl])

        combine(ia_h, bc)
        pack_bits(bc, o_ahb)

    return k(h_pf, h_pp, h_ia)


def _reach(ap_e, plbits, ahbits):
    out_type = [jax.ShapeDtypeStruct((_NC, _NS, 16), _f32)]
    scratch = [
        pltpu.VMEM((_NBW,), _i32),
        pltpu.VMEM((_NBW,), _i32),
        pltpu.VMEM((_CB,), _i32),
        pltpu.VMEM((_CB,), _i32),
        pltpu.VMEM((16,), _f32),
    ]

    @functools.partial(pl.kernel, out_type=out_type, mesh=_MESH,
                       scratch_types=scratch, compiler_params=_SC_PARAMS)
    def k(ap_h, plb_h, ahb_h, o_rch, ahb_v, plb_v, a_v, p_v, out_v):
        c, s, wid = _wid()
        pltpu.sync_copy(ahb_h, ahb_v)
        pltpu.sync_copy(plb_h, plb_v)

        def bit_at(tab, idx16):
            w = plsc.load_gather(tab, [lax.shift_right_logical(idx16, 5)])
            return lax.shift_right_logical(w, idx16 & 31) & 1

        def body(i, acc):
            blk = wid + i * _NW
            sle = pl.ds(blk * _CB, _CB)
            pltpu.sync_copy(ap_h.at[0, sle], a_v)
            pltpu.sync_copy(ap_h.at[1, sle], p_v)
            for j in range(_CB // 16):
                sl16 = pl.ds(j * 16, 16)
                hit = bit_at(ahb_v, a_v[sl16]) & bit_at(plb_v, p_v[sl16])
                acc = acc + hit.astype(_f32)
            return acc

        acc = lax.fori_loop(0, _nblk_for(wid, _NBLK_BIG), body,
                            jnp.zeros((16,), _f32))
        out_v[...] = acc
        pltpu.sync_copy(out_v, o_rch.at[c, s])

    return k(ap_e, plbits, ahbits)[0]


_RB = 10000


def _modulate(x, pb_pad, reach_part):
    def body(x_ref, pb_ref, pb2_ref, rch_ref, o_ref, s_ref):
        i = pl.program_id(0)

        @pl.when(i == 0)
        def _():
            ps = jnp.sum(pb2_ref[...])
            rs = jnp.sum(rch_ref[...])
            s_ref[0] = 1.0 / jnp.maximum(ps, 1e-12)
            s_ref[1] = (jnp.where(ps > 0, 1.0, 0.0)
                        + jnp.where(rs > 0, 1.0, 0.0))

        o_ref[...] = x_ref[...] * (pb_ref[...] * s_ref[0]) + s_ref[1]

    pb2 = pb_pad.reshape(_NP_P // 128, 128)
    rch = reach_part.reshape(_NC * _NS, 16)
    return pl.pallas_call(
        body,
        grid=(N_PAPER // _RB,),
        in_specs=[
            pl.BlockSpec((_RB, D_FEAT), lambda i: (i, 0)),
            pl.BlockSpec((_RB, 1), lambda i: (i, 0)),
            pl.BlockSpec(pb2.shape, lambda i: (0, 0)),
            pl.BlockSpec(rch.shape, lambda i: (0, 0)),
        ],
        out_specs=pl.BlockSpec((_RB, D_FEAT), lambda i: (i, 0)),
        out_shape=jax.ShapeDtypeStruct((N_PAPER, D_FEAT), jnp.float32),
        scratch_shapes=[pltpu.SMEM((2,), jnp.float32)],
    )(x, pb_pad[:N_PAPER].reshape(N_PAPER, 1), pb2, rch)


def kernel(x_paper, edge_inst_auth, edge_auth_paper, edge_cite, edge_paper_field):
    (h_pf,) = _hist_pf(edge_paper_field)
    h_pp1, h_ia1 = _hist_ppia(edge_cite, edge_inst_auth)
    pb_pad, plbits, ahbits = _build(h_pf, h_pp1, h_ia1)
    reach_part = _reach(edge_auth_paper, plbits, ahbits)
    return _modulate(x_paper, pb_pad, reach_part)


# speedup vs baseline: 1.7941x; 1.0138x over previous
import functools

import jax
import jax.numpy as jnp
from jax import lax
from jax.experimental import pallas as pl
from jax.experimental.pallas import tpu as pltpu
from jax.experimental.pallas import tpu_sc as plsc

N_INST = 10000
N_AUTH = 100000
N_PAPER = 100000
N_FIELD = 50000
E_IA = 400000
E_AP = 1600000
E_PP = 1600000
E_PF = 1600000
D_FEAT = 128

_NC, _NS = 2, 16
_NW = _NC * _NS
_CB = 3200
_NBLK_BIG = E_PP // _CB
_NBLK_IA = E_IA // _CB

_NP_P = 102400
_SL_P = _NP_P // _NS
_NBW = _NP_P // 32
_WPT = _NBW // _NS

_MESH = plsc.VectorSubcoreMesh(
    core_axis_name="c", subcore_axis_name="s", num_cores=_NC, num_subcores=_NS)
_SC_PARAMS = pltpu.CompilerParams(
    use_tc_tiling_on_sc=False, needs_layout_passes=False)

_f32 = jnp.float32
_i32 = jnp.int32


def _wid():
    c = lax.axis_index("c")
    s = lax.axis_index("s")
    return c, s, s * _NC + c


def _nblk_for(wid, nblk):
    return (nblk - wid + _NW - 1) // _NW


def _hist_rows(specs):
    n = len(specs)
    out_type = [jax.ShapeDtypeStruct((_NC, _NP_P), _f32) for _ in range(n)]
    scratch = ([pltpu.VMEM((_CB,), _i32), pltpu.VMEM((128,), _f32)]
               + [pltpu.VMEM_SHARED((_NP_P,), _f32) for _ in range(n)]
               + [pltpu.SemaphoreType.DMA])

    @functools.partial(pl.kernel, out_type=out_type, mesh=_MESH,
                       scratch_types=scratch, compiler_params=_SC_PARAMS)
    def k(*refs):
        e_hs = refs[:n]
        ones_h, zeros_h = refs[n], refs[n + 1]
        outs = refs[n + 2:n + 2 + n]
        idx_v, ones_v = refs[n + 2 + n], refs[n + 3 + n]
        accs = refs[n + 4 + n:n + 4 + 2 * n]
        sem = refs[-1]
        c, s, wid = _wid()
        sl = pl.ds(s * _SL_P, _SL_P)
        pltpu.sync_copy(ones_h, ones_v)
        for acc in accs:
            pltpu.sync_copy(zeros_h, acc.at[sl])
        plsc.subcore_barrier()

        for (row, nblk), e_h, acc in zip(specs, e_hs, accs):
            def body(i, carry, e_h=e_h, acc=acc, row=row):
                blk = wid + i * _NW
                pltpu.sync_copy(e_h.at[row, pl.ds(blk * _CB, _CB)], idx_v)
                cps = [pltpu.async_copy(
                    ones_v, acc.at[idx_v.at[pl.ds(j * 128, 128)]], sem,
                    add=True)
                       for j in range(_CB // 128)]
                for cp in cps:
                    cp.wait()
                return carry
            lax.fori_loop(0, _nblk_for(wid, nblk), body, 0)

        plsc.subcore_barrier()
        for acc, o in zip(accs, outs):
            pltpu.sync_copy(acc.at[sl], o.at[c, sl])

    ones = jnp.ones((128,), _f32)
    zeros = jnp.zeros((_SL_P,), _f32)

    def call(*edge_arrays):
        return k(*edge_arrays, ones, zeros)
    return call


_hist_pf = _hist_rows([(0, _NBLK_BIG)])
_hist_ppia = _hist_rows([(1, _NBLK_BIG), (1, _NBLK_IA)])


def _build(h_pf, h_pp, h_ia):
    out_type = [
        jax.ShapeDtypeStruct((_NP_P,), _f32),
        jax.ShapeDtypeStruct((_NBW,), _i32),
        jax.ShapeDtypeStruct((_NBW,), _i32),
    ]
    scratch = [
        pltpu.VMEM((_SL_P,), _f32),
        pltpu.VMEM((_SL_P,), _f32),
        pltpu.VMEM((_SL_P,), _f32),
        pltpu.VMEM((_WPT,), _i32),
    ]

    @functools.partial(pl.kernel, out_type=out_type, mesh=_MESH,
                       scratch_types=scratch, compiler_params=_SC_PARAMS)
    def k(pf_h, pp_h, ia_h, o_pb, o_plb, o_ahb, b0, b1, bc, wv):
        c, s, wid = _wid()
        sl = pl.ds(s * _SL_P, _SL_P)
        lanes = lax.iota(_i32, 16)

        def combine(h2, dst):
            pltpu.sync_copy(h2.at[0, sl], b0)
            pltpu.sync_copy(h2.at[1, sl], b1)

            def body(i, carry):
                i16 = pl.ds(i * 16, 16)
                dst[i16] = b0[i16] + b1[i16]
                return carry
            lax.fori_loop(0, _SL_P // 16, body, 0)

        def pack_bits(src, o_bits):
            for base in [*range(0, _WPT - 15, 16), _WPT - 16]:
                cur = jnp.zeros((16,), _i32)
                for t in range(16):
                    w = base + t
                    lo = (src[pl.ds(w * 32, 16)] > 0).astype(_i32) << lanes
                    hi = (src[pl.ds(w * 32 + 16, 16)] > 0).astype(_i32) << lanes
                    wd = jnp.sum(lo) | (jnp.sum(hi) << 16)
                    cur = jnp.where(lanes == t, wd, cur)
                wv[pl.ds(base, 16)] = cur

            @pl.when(c == 0)
            def _():
                pltpu.sync_copy(wv, o_bits.at[pl.ds(s * _WPT, _WPT)])

        combine(pf_h, bc)
        pack_bits(bc, o_plb)
        pltpu.sync_copy(pp_h.at[0, sl], b0)
        pltpu.sync_copy(pp_h.at[1, sl], b1)

        def pb_body(i, carry):
            i16 = pl.ds(i * 16, 16)
            pos = (bc[i16] > 0).astype(_f32)
            b0[i16] = pos * (b0[i16] + b1[i16])
            return carry
        lax.fori_loop(0, _SL_P // 16, pb_body, 0)

        @pl.when(c == 0)
        def _():
            pltpu.sync_copy(b0, o_pb.at[sl])

        combine(ia_h, bc)
        pack_bits(bc, o_ahb)

    return k(h_pf, h_pp, h_ia)


def _reach(ap_e, plbits, ahbits):
    out_type = [jax.ShapeDtypeStruct((_NC, _NS, 16), _f32)]
    scratch = [
        pltpu.VMEM((_NBW,), _i32),
        pltpu.VMEM((_NBW,), _i32),
        pltpu.VMEM((_CB,), _i32),
        pltpu.VMEM((_CB,), _i32),
        pltpu.VMEM((16,), _f32),
    ]

    @functools.partial(pl.kernel, out_type=out_type, mesh=_MESH,
                       scratch_types=scratch, compiler_params=_SC_PARAMS)
    def k(ap_h, plb_h, ahb_h, o_rch, ahb_v, plb_v, a_v, p_v, out_v):
        c, s, wid = _wid()
        pltpu.sync_copy(ahb_h, ahb_v)
        pltpu.sync_copy(plb_h, plb_v)

        def bit_at(tab, idx16):
            w = plsc.load_gather(tab, [lax.shift_right_logical(idx16, 5)])
            return lax.shift_right_logical(w, idx16 & 31) & 1

        nw = _nblk_for(wid, _NBLK_BIG)

        def wcond(carry):
            i, acc = carry
            return jnp.logical_and(i < nw, jnp.sum(acc) <= 0.0)

        def wbody(carry):
            i, acc = carry
            blk = wid + i * _NW
            sle = pl.ds(blk * _CB, _CB)
            pltpu.sync_copy(ap_h.at[0, sle], a_v)
            pltpu.sync_copy(ap_h.at[1, sle], p_v)
            for j in range(_CB // 16):
                sl16 = pl.ds(j * 16, 16)
                hit = bit_at(ahb_v, a_v[sl16]) & bit_at(plb_v, p_v[sl16])
                acc = acc + hit.astype(_f32)
            return i + 1, acc

        _, acc = lax.while_loop(wcond, wbody,
                                (jnp.int32(0), jnp.zeros((16,), _f32)))
        out_v[...] = acc
        pltpu.sync_copy(out_v, o_rch.at[c, s])

    return k(ap_e, plbits, ahbits)[0]


_RB = 10000


def _modulate(x, pb_pad, reach_part):
    def body(x_ref, pb_ref, pb2_ref, rch_ref, o_ref, s_ref):
        i = pl.program_id(0)

        @pl.when(i == 0)
        def _():
            ps = jnp.sum(pb2_ref[...])
            rs = jnp.sum(rch_ref[...])
            s_ref[0] = 1.0 / jnp.maximum(ps, 1e-12)
            s_ref[1] = (jnp.where(ps > 0, 1.0, 0.0)
                        + jnp.where(rs > 0, 1.0, 0.0))

        o_ref[...] = x_ref[...] * (pb_ref[...] * s_ref[0]) + s_ref[1]

    pb2 = pb_pad.reshape(_NP_P // 128, 128)
    rch = reach_part.reshape(_NC * _NS, 16)
    return pl.pallas_call(
        body,
        grid=(N_PAPER // _RB,),
        in_specs=[
            pl.BlockSpec((_RB, D_FEAT), lambda i: (i, 0)),
            pl.BlockSpec((_RB, 1), lambda i: (i, 0)),
            pl.BlockSpec(pb2.shape, lambda i: (0, 0)),
            pl.BlockSpec(rch.shape, lambda i: (0, 0)),
        ],
        out_specs=pl.BlockSpec((_RB, D_FEAT), lambda i: (i, 0)),
        out_shape=jax.ShapeDtypeStruct((N_PAPER, D_FEAT), jnp.float32),
        scratch_shapes=[pltpu.SMEM((2,), jnp.float32)],
    )(x, pb_pad[:N_PAPER].reshape(N_PAPER, 1), pb2, rch)


def kernel(x_paper, edge_inst_auth, edge_auth_paper, edge_cite, edge_paper_field):
    (h_pf,) = _hist_pf(edge_paper_field)
    h_pp1, h_ia1 = _hist_ppia(edge_cite, edge_inst_auth)
    pb_pad, plbits, ahbits = _build(h_pf, h_pp1, h_ia1)
    reach_part = _reach(edge_auth_paper, plbits, ahbits)
    return _modulate(x_paper, pb_pad, reach_part)


# speedup vs baseline: 1.7967x; 1.0015x over previous
import functools

import jax
import jax.numpy as jnp
from jax import lax
from jax.experimental import pallas as pl
from jax.experimental.pallas import tpu as pltpu
from jax.experimental.pallas import tpu_sc as plsc

N_INST = 10000
N_AUTH = 100000
N_PAPER = 100000
N_FIELD = 50000
E_IA = 400000
E_AP = 1600000
E_PP = 1600000
E_PF = 1600000
D_FEAT = 128

_NC, _NS = 2, 16
_NW = _NC * _NS
_CB = 3200
_NBLK_BIG = E_PP // _CB
_NBLK_IA = E_IA // _CB

_NP_P = 102400
_SL_P = _NP_P // _NS
_NBW = _NP_P // 32
_WPT = _NBW // _NS

def _mesh():
    return plsc.VectorSubcoreMesh(core_axis_name="c", subcore_axis_name="s",
                                  num_cores=_NC, num_subcores=_NS)


_SC_PARAMS = pltpu.CompilerParams(
    use_tc_tiling_on_sc=False, needs_layout_passes=False)

_f32 = jnp.float32
_i32 = jnp.int32


def _wid():
    c = lax.axis_index("c")
    s = lax.axis_index("s")
    return c, s, s * _NC + c


def _nblk_for(wid, nblk):
    return (nblk - wid + _NW - 1) // _NW


def _hist_rows(specs):
    n = len(specs)
    out_type = [jax.ShapeDtypeStruct((_NC, _NP_P), _f32) for _ in range(n)]
    scratch = ([pltpu.VMEM((_CB,), _i32), pltpu.VMEM((128,), _f32)]
               + [pltpu.VMEM_SHARED((_NP_P,), _f32) for _ in range(n)]
               + [pltpu.SemaphoreType.DMA])

    @functools.partial(pl.kernel, out_type=out_type, mesh=_mesh(),
                       scratch_types=scratch, compiler_params=_SC_PARAMS)
    def k(*refs):
        e_hs = refs[:n]
        ones_h, zeros_h = refs[n], refs[n + 1]
        outs = refs[n + 2:n + 2 + n]
        idx_v, ones_v = refs[n + 2 + n], refs[n + 3 + n]
        accs = refs[n + 4 + n:n + 4 + 2 * n]
        sem = refs[-1]
        c, s, wid = _wid()
        sl = pl.ds(s * _SL_P, _SL_P)
        pltpu.sync_copy(ones_h, ones_v)
        for acc in accs:
            pltpu.sync_copy(zeros_h, acc.at[sl])
        plsc.subcore_barrier()

        for (row, nblk), e_h, acc in zip(specs, e_hs, accs):
            def body(i, carry, e_h=e_h, acc=acc, row=row):
                blk = wid + i * _NW
                pltpu.sync_copy(e_h.at[row, pl.ds(blk * _CB, _CB)], idx_v)
                cps = [pltpu.async_copy(
                    ones_v, acc.at[idx_v.at[pl.ds(j * 128, 128)]], sem,
                    add=True)
                       for j in range(_CB // 128)]
                for cp in cps:
                    cp.wait()
                return carry
            lax.fori_loop(0, _nblk_for(wid, nblk), body, 0)

        plsc.subcore_barrier()
        for acc, o in zip(accs, outs):
            pltpu.sync_copy(acc.at[sl], o.at[c, sl])

    ones = jnp.ones((128,), _f32)
    zeros = jnp.zeros((_SL_P,), _f32)

    def call(*edge_arrays):
        return k(*edge_arrays, ones, zeros)
    return call


def _build(h_pf, h_pp, h_ia):
    out_type = [
        jax.ShapeDtypeStruct((_NP_P,), _f32),
        jax.ShapeDtypeStruct((_NBW,), _i32),
        jax.ShapeDtypeStruct((_NBW,), _i32),
    ]
    scratch = [
        pltpu.VMEM((_SL_P,), _f32),
        pltpu.VMEM((_SL_P,), _f32),
        pltpu.VMEM((_SL_P,), _f32),
        pltpu.VMEM((_WPT,), _i32),
    ]

    @functools.partial(pl.kernel, out_type=out_type, mesh=_mesh(),
                       scratch_types=scratch, compiler_params=_SC_PARAMS)
    def k(pf_h, pp_h, ia_h, o_pb, o_plb, o_ahb, b0, b1, bc, wv):
        c, s, wid = _wid()
        sl = pl.ds(s * _SL_P, _SL_P)
        lanes = lax.iota(_i32, 16)

        def combine(h2, dst):
            pltpu.sync_copy(h2.at[0, sl], b0)
            pltpu.sync_copy(h2.at[1, sl], b1)

            def body(i, carry):
                i16 = pl.ds(i * 16, 16)
                dst[i16] = b0[i16] + b1[i16]
                return carry
            lax.fori_loop(0, _SL_P // 16, body, 0)

        def pack_bits(src, o_bits):
            for base in [*range(0, _WPT - 15, 16), _WPT - 16]:
                cur = jnp.zeros((16,), _i32)
                for t in range(16):
                    w = base + t
                    lo = (src[pl.ds(w * 32, 16)] > 0).astype(_i32) << lanes
                    hi = (src[pl.ds(w * 32 + 16, 16)] > 0).astype(_i32) << lanes
                    wd = jnp.sum(lo) | (jnp.sum(hi) << 16)
                    cur = jnp.where(lanes == t, wd, cur)
                wv[pl.ds(base, 16)] = cur

            @pl.when(c == 0)
            def _():
                pltpu.sync_copy(wv, o_bits.at[pl.ds(s * _WPT, _WPT)])

        combine(pf_h, bc)
        pack_bits(bc, o_plb)
        pltpu.sync_copy(pp_h.at[0, sl], b0)
        pltpu.sync_copy(pp_h.at[1, sl], b1)

        def pb_body(i, carry):
            i16 = pl.ds(i * 16, 16)
            pos = (bc[i16] > 0).astype(_f32)
            b0[i16] = pos * (b0[i16] + b1[i16])
            return carry
        lax.fori_loop(0, _SL_P // 16, pb_body, 0)

        @pl.when(c == 0)
        def _():
            pltpu.sync_copy(b0, o_pb.at[sl])

        combine(ia_h, bc)
        pack_bits(bc, o_ahb)

    return k(h_pf, h_pp, h_ia)


def _reach(ap_e, plbits, ahbits):
    out_type = [jax.ShapeDtypeStruct((_NC, _NS, 16), _f32)]
    scratch = [
        pltpu.VMEM((_NBW,), _i32),
        pltpu.VMEM((_NBW,), _i32),
        pltpu.VMEM((_CB,), _i32),
        pltpu.VMEM((_CB,), _i32),
        pltpu.VMEM((16,), _f32),
    ]

    @functools.partial(pl.kernel, out_type=out_type, mesh=_mesh(),
                       scratch_types=scratch, compiler_params=_SC_PARAMS)
    def k(ap_h, plb_h, ahb_h, o_rch, ahb_v, plb_v, a_v, p_v, out_v):
        c, s, wid = _wid()
        pltpu.sync_copy(ahb_h, ahb_v)
        pltpu.sync_copy(plb_h, plb_v)

        def bit_at(tab, idx16):
            w = plsc.load_gather(tab, [lax.shift_right_logical(idx16, 5)])
            return lax.shift_right_logical(w, idx16 & 31) & 1

        nw = _nblk_for(wid, _NBLK_BIG)

        def wcond(carry):
            i, acc = carry
            return jnp.logical_and(i < nw, jnp.sum(acc) <= 0.0)

        def wbody(carry):
            i, acc = carry
            blk = wid + i * _NW
            sle = pl.ds(blk * _CB, _CB)
            pltpu.sync_copy(ap_h.at[0, sle], a_v)
            pltpu.sync_copy(ap_h.at[1, sle], p_v)
            for j in range(_CB // 16):
                sl16 = pl.ds(j * 16, 16)
                hit = bit_at(ahb_v, a_v[sl16]) & bit_at(plb_v, p_v[sl16])
                acc = acc + hit.astype(_f32)
            return i + 1, acc

        _, acc = lax.while_loop(wcond, wbody,
                                (jnp.int32(0), jnp.zeros((16,), _f32)))
        out_v[...] = acc
        pltpu.sync_copy(out_v, o_rch.at[c, s])

    return k(ap_e, plbits, ahbits)[0]


_RB = 10000


def _modulate(x, pb_pad, reach_part):
    def body(x_ref, pb_ref, pb2_ref, rch_ref, o_ref, s_ref):
        i = pl.program_id(0)

        @pl.when(i == 0)
        def _():
            ps = jnp.sum(pb2_ref[...])
            rs = jnp.sum(rch_ref[...])
            s_ref[0] = 1.0 / jnp.maximum(ps, 1e-12)
            s_ref[1] = (jnp.where(ps > 0, 1.0, 0.0)
                        + jnp.where(rs > 0, 1.0, 0.0))

        o_ref[...] = x_ref[...] * (pb_ref[...] * s_ref[0]) + s_ref[1]

    pb2 = pb_pad.reshape(_NP_P // 128, 128)
    rch = reach_part.reshape(_NC * _NS, 16)
    return pl.pallas_call(
        body,
        grid=(N_PAPER // _RB,),
        in_specs=[
            pl.BlockSpec((_RB, D_FEAT), lambda i: (i, 0)),
            pl.BlockSpec((_RB, 1), lambda i: (i, 0)),
            pl.BlockSpec(pb2.shape, lambda i: (0, 0)),
            pl.BlockSpec(rch.shape, lambda i: (0, 0)),
        ],
        out_specs=pl.BlockSpec((_RB, D_FEAT), lambda i: (i, 0)),
        out_shape=jax.ShapeDtypeStruct((N_PAPER, D_FEAT), jnp.float32),
        scratch_shapes=[pltpu.SMEM((2,), jnp.float32)],
    )(x, pb_pad[:N_PAPER].reshape(N_PAPER, 1), pb2, rch)


def kernel(x_paper, edge_inst_auth, edge_auth_paper, edge_cite, edge_paper_field):
    (h_pf,) = _hist_rows([(0, _NBLK_BIG)])(edge_paper_field)
    h_pp1, h_ia1 = _hist_rows([(1, _NBLK_BIG), (1, _NBLK_IA)])(
        edge_cite, edge_inst_auth)
    pb_pad, plbits, ahbits = _build(h_pf, h_pp1, h_ia1)
    reach_part = _reach(edge_auth_paper, plbits, ahbits)
    return _modulate(x_paper, pb_pad, reach_part)
